# Initial kernel scaffold; baseline (speedup 1.0000x reference)
import jax
import jax.numpy as jnp
from jax.experimental import pallas as pl


def kernel(user_ids, item_seq, last_pos, user_table, item_table, global_user_emb):
    raise NotImplementedError("write your pallas kernel here")


# speedup vs baseline: 6.8204x; 6.8204x over previous
import functools

import jax
import jax.numpy as jnp
from jax import lax
from jax.experimental import pallas as pl
from jax.experimental.pallas import tpu as pltpu
from jax.experimental.pallas import tpu_sc as plsc

BATCH = 4096
HIST = 50
EMBED_DIM = 64
LANES = 16
NUM_WORKERS = 32
B_PER_W = BATCH // NUM_WORKERS


def _sc_kernel(user_ids_hbm, item_seq_hbm, last_pos_hbm,
               user_table_hbm, item_table_hbm, global_emb_hbm,
               out_hbm,
               uid_v, lp_v, seq_v, iid_v, urows_v, irows_v, out_v, g_v,
               sem_u, sem_i):
    num_cores = 2
    wid = lax.axis_index("s") * num_cores + lax.axis_index("c")
    base = wid * B_PER_W

    pltpu.sync_copy(user_ids_hbm.at[pl.ds(base, B_PER_W)], uid_v)
    u_gather = pltpu.async_copy(user_table_hbm.at[uid_v], urows_v, sem_u)

    pltpu.sync_copy(last_pos_hbm.at[pl.ds(base, B_PER_W)], lp_v)
    pltpu.sync_copy(item_seq_hbm.at[pl.ds(base * HIST, B_PER_W * HIST)],
                    seq_v.at[pl.ds(0, B_PER_W * HIST)])
    pltpu.sync_copy(global_emb_hbm, g_v)

    lane_ids = lax.iota(jnp.int32, LANES)

    def pick(g, carry):
        lpc = lp_v[pl.ds(g * LANES, LANES)]
        vals = jnp.zeros((LANES,), jnp.int32)
        for i in range(LANES):
            pos = (g * LANES + i) * HIST + lpc[i]
            iid_s = seq_v[pl.ds(pos, LANES)][0]
            vals = jnp.where(lane_ids == i, jnp.full((LANES,), iid_s), vals)
        iid_v[pl.ds(g * LANES, LANES)] = vals
        return carry

    lax.fori_loop(0, B_PER_W // LANES, pick, 0)

    i_gather = pltpu.async_copy(item_table_hbm.at[iid_v], irows_v, sem_i)

    g_chunks = [g_v[pl.ds(c * LANES, LANES)] for c in range(EMBED_DIM // LANES)]

    u_gather.wait()
    i_gather.wait()

    one = jnp.float32(1.0)
    zero = jnp.float32(0.0)

    def body(g, carry):
        su = jnp.where(uid_v[pl.ds(g * LANES, LANES)] != 0, one, zero)
        si = jnp.where(iid_v[pl.ds(g * LANES, LANES)] != 0, one, zero)
        for i in range(LANES):
            e = g * LANES + i
            s_u = jnp.full((LANES,), su[i], dtype=jnp.float32)
            s_i = jnp.full((LANES,), si[i], dtype=jnp.float32)
            for c in range(EMBED_DIM // LANES):
                u = urows_v[e, pl.ds(c * LANES, LANES)]
                it = irows_v[e, pl.ds(c * LANES, LANES)]
                out_v[e, pl.ds(c * LANES, LANES)] = (
                    u * s_u + it * s_i + g_chunks[c])
        return carry

    lax.fori_loop(0, B_PER_W // LANES, body, 0)

    pltpu.sync_copy(out_v, out_hbm.at[pl.ds(base, B_PER_W), :])


@jax.jit
def _run(user_ids, item_seq, last_pos, user_table, item_table, global_user_emb):
    mesh = plsc.VectorSubcoreMesh(core_axis_name="c", subcore_axis_name="s")
    f = functools.partial(
        pl.kernel,
        mesh=mesh,
        compiler_params=pltpu.CompilerParams(use_tc_tiling_on_sc=False),
        out_type=jax.ShapeDtypeStruct((BATCH, EMBED_DIM), jnp.float32),
        scratch_types=[
            pltpu.VMEM((B_PER_W,), jnp.int32),
            pltpu.VMEM((B_PER_W,), jnp.int32),
            pltpu.VMEM((B_PER_W * HIST + LANES,), jnp.int32),
            pltpu.VMEM((B_PER_W,), jnp.int32),
            pltpu.VMEM((B_PER_W, EMBED_DIM), jnp.float32),
            pltpu.VMEM((B_PER_W, EMBED_DIM), jnp.float32),
            pltpu.VMEM((B_PER_W, EMBED_DIM), jnp.float32),
            pltpu.VMEM((EMBED_DIM,), jnp.float32),
            pltpu.SemaphoreType.DMA,
            pltpu.SemaphoreType.DMA,
        ],
    )(_sc_kernel)
    return f(user_ids.astype(jnp.int32),
             item_seq.astype(jnp.int32).reshape(-1),
             last_pos.astype(jnp.int32), user_table, item_table,
             global_user_emb)


def kernel(user_ids, item_seq, last_pos, user_table, item_table, global_user_emb):
    return _run(user_ids, item_seq, last_pos, user_table, item_table,
                global_user_emb)


# speedup vs baseline: 9.0835x; 1.3318x over previous
import functools

import jax
import jax.numpy as jnp
from jax import lax
from jax.experimental import pallas as pl
from jax.experimental.pallas import tpu as pltpu
from jax.experimental.pallas import tpu_sc as plsc

BATCH = 4096
HIST = 50
EMBED_DIM = 64
LANES = 16
NUM_WORKERS = 32
B_PER_W = BATCH // NUM_WORKERS
GROUPS = B_PER_W // LANES
ROW_BYTES = EMBED_DIM * 4


def _sc_kernel(user_ids_hbm, item_seq_hbm, last_pos_hbm,
               user_table_hbm, item_table_hbm, global_emb_hbm,
               out_hbm,
               uid_v, lp_v, seq_v, iid_v, urows_v, irows_v, out_v, g_v,
               sem_u, sem_i):
    num_cores = 2
    wid = lax.axis_index("s") * num_cores + lax.axis_index("c")
    base = wid * B_PER_W

    pltpu.sync_copy(user_ids_hbm.at[pl.ds(base, B_PER_W)], uid_v)
    pltpu.sync_copy(last_pos_hbm.at[pl.ds(base, B_PER_W)], lp_v)
    pltpu.sync_copy(item_seq_hbm.at[pl.ds(base * HIST, B_PER_W * HIST)],
                    seq_v.at[pl.ds(0, B_PER_W * HIST)])
    pltpu.sync_copy(global_emb_hbm, g_v)

    lane_ids = lax.iota(jnp.int32, LANES)

    def fire(g, carry):
        gbase = g * LANES
        uc = uid_v[pl.ds(gbase, LANES)]
        lpc = lp_v[pl.ds(gbase, LANES)]
        vals = jnp.zeros((LANES,), jnp.int32)
        for i in range(LANES):
            e = gbase + i
            uid_s = uc[i]
            pltpu.async_copy(user_table_hbm.at[uid_s], urows_v.at[e], sem_u)
            pos = e * HIST + lpc[i]
            iid_s = seq_v[pl.ds(pos, LANES)][0]
            pltpu.async_copy(item_table_hbm.at[iid_s], irows_v.at[e], sem_i)
            vals = jnp.where(lane_ids == i, jnp.full((LANES,), iid_s), vals)
        iid_v[pl.ds(gbase, LANES)] = vals
        return carry

    lax.fori_loop(0, GROUPS, fire, 0)

    pltpu.make_async_copy(
        user_table_hbm.at[pl.ds(0, B_PER_W), :], urows_v, sem_u).wait()
    pltpu.make_async_copy(
        item_table_hbm.at[pl.ds(0, B_PER_W), :], irows_v, sem_i).wait()

    g_chunks = [g_v[pl.ds(c * LANES, LANES)] for c in range(EMBED_DIM // LANES)]

    one = jnp.float32(1.0)
    zero = jnp.float32(0.0)

    def body(g, carry):
        su = jnp.where(uid_v[pl.ds(g * LANES, LANES)] != 0, one, zero)
        si = jnp.where(iid_v[pl.ds(g * LANES, LANES)] != 0, one, zero)
        for i in range(LANES):
            e = g * LANES + i
            s_u = jnp.full((LANES,), su[i], dtype=jnp.float32)
            s_i = jnp.full((LANES,), si[i], dtype=jnp.float32)
            for c in range(EMBED_DIM // LANES):
                u = urows_v[e, pl.ds(c * LANES, LANES)]
                it = irows_v[e, pl.ds(c * LANES, LANES)]
                out_v[e, pl.ds(c * LANES, LANES)] = (
                    u * s_u + it * s_i + g_chunks[c])
        return carry

    lax.fori_loop(0, GROUPS, body, 0)

    pltpu.sync_copy(out_v, out_hbm.at[pl.ds(base, B_PER_W), :])


@jax.jit
def _run(user_ids, item_seq, last_pos, user_table, item_table, global_user_emb):
    mesh = plsc.VectorSubcoreMesh(core_axis_name="c", subcore_axis_name="s")
    f = functools.partial(
        pl.kernel,
        mesh=mesh,
        out_type=jax.ShapeDtypeStruct((BATCH, EMBED_DIM), jnp.float32),
        scratch_types=[
            pltpu.VMEM((B_PER_W,), jnp.int32),
            pltpu.VMEM((B_PER_W,), jnp.int32),
            pltpu.VMEM((B_PER_W * HIST + LANES,), jnp.int32),
            pltpu.VMEM((B_PER_W,), jnp.int32),
            pltpu.VMEM((B_PER_W, EMBED_DIM), jnp.float32),
            pltpu.VMEM((B_PER_W, EMBED_DIM), jnp.float32),
            pltpu.VMEM((B_PER_W, EMBED_DIM), jnp.float32),
            pltpu.VMEM((EMBED_DIM,), jnp.float32),
            pltpu.SemaphoreType.DMA,
            pltpu.SemaphoreType.DMA,
        ],
    )(_sc_kernel)
    return f(user_ids.astype(jnp.int32),
             item_seq.astype(jnp.int32).reshape(-1),
             last_pos.astype(jnp.int32), user_table, item_table,
             global_user_emb)


def kernel(user_ids, item_seq, last_pos, user_table, item_table, global_user_emb):
    return _run(user_ids, item_seq, last_pos, user_table, item_table,
                global_user_emb)
